# baseline (device time: 12546 ns/iter reference)
import jax
import jax.numpy as jnp
from jax import lax
from jax.experimental import pallas as pl
from jax.experimental.pallas import tpu as pltpu

N_DEV = 4
DISTS = (2, 1, 3)


def kernel(x, w_mat):
    m_per, k = x.shape
    _, n = w_mat.shape
    n_per = n // N_DEV

    def body(x_ref, w_ref, out_hbm, out_stage, send_buf, recv_buf,
             send_sems, recv_sems, ready_sems, out_copy_sem):
        my = lax.axis_index("i")

        barrier_sem = pltpu.get_barrier_semaphore()
        pl.semaphore_signal(
            barrier_sem, inc=1,
            device_id=(my,), device_id_type=pl.DeviceIdType.MESH,
        )
        pl.semaphore_wait(barrier_sem, 1)

        for i, d in enumerate(DISTS):
            pl.semaphore_signal(
                ready_sems.at[i], inc=1,
                device_id=((my - d) % N_DEV,),
                device_id_type=pl.DeviceIdType.MESH,
            )

        rdmas = []
        for i, d in enumerate(DISTS):
            tgt = (my + d) % N_DEV
            yc = jnp.dot(
                x_ref[:, :], w_ref[:, pl.ds(tgt * n_per, n_per)],
                preferred_element_type=jnp.float32,
            )
            send_buf[i, :, :] = (yc * jax.nn.sigmoid(yc)).astype(jnp.bfloat16)
            pl.semaphore_wait(ready_sems.at[i], 1)
            rdma = pltpu.make_async_remote_copy(
                src_ref=send_buf.at[i],
                dst_ref=recv_buf.at[i],
                send_sem=send_sems.at[i],
                recv_sem=recv_sems.at[i],
                device_id=(tgt,),
                device_id_type=pl.DeviceIdType.MESH,
            )
            rdma.start()
            rdmas.append(rdma)

        yc = jnp.dot(
            x_ref[:, :], w_ref[:, pl.ds(my * n_per, n_per)],
            preferred_element_type=jnp.float32,
        )
        out_stage[pl.ds(my * m_per, m_per), :] = yc * jax.nn.sigmoid(yc)

        for i in (1, 2, 0):
            rdmas[i].wait_recv()
            src = (my - DISTS[i]) % N_DEV
            out_stage[pl.ds(src * m_per, m_per), :] = recv_buf[i, :, :].astype(
                jnp.float32
            )
        for rdma in rdmas:
            rdma.wait_send()
        out_cp = pltpu.make_async_copy(out_stage, out_hbm, out_copy_sem)
        out_cp.start()
        out_cp.wait()

    out_shape = jax.ShapeDtypeStruct((N_DEV * m_per, n_per), jnp.float32)
    return pl.pallas_call(
        body,
        out_shape=out_shape,
        in_specs=[
            pl.BlockSpec(memory_space=pltpu.VMEM),
            pl.BlockSpec(memory_space=pltpu.VMEM),
        ],
        out_specs=pl.BlockSpec(memory_space=pl.ANY),
        scratch_shapes=[
            pltpu.VMEM((N_DEV * m_per, n_per), jnp.float32),
            pltpu.VMEM((N_DEV - 1, m_per, n_per), jnp.bfloat16),
            pltpu.VMEM((N_DEV - 1, m_per, n_per), jnp.bfloat16),
            pltpu.SemaphoreType.DMA((N_DEV - 1,)),
            pltpu.SemaphoreType.DMA((N_DEV - 1,)),
            pltpu.SemaphoreType.REGULAR((N_DEV - 1,)),
            pltpu.SemaphoreType.DMA,
        ],
        compiler_params=pltpu.CompilerParams(collective_id=0),
    )(x, w_mat)
